# initial kernel scaffold (unmeasured)
import jax
import jax.numpy as jnp
from jax import lax
from jax.experimental import pallas as pl
from jax.experimental.pallas import tpu as pltpu

N_DEV = 4


def kernel(x, w_mat, scale_x, scale_w):
    m_per, k = x.shape
    _, n = w_mat.shape
    n_per = n // N_DEV

    me = lax.axis_index("i")
    w_blk = lax.dynamic_slice(w_mat, (0, me * n_per), (k, n_per))
    x8 = x.astype(jnp.float8_e4m3fn)
    w8 = w_blk.astype(jnp.float8_e4m3fn)
    sx = scale_x.astype(jnp.float32)
    sw = scale_w.astype(jnp.float32)

    def body(x_ref, w_ref, sx_ref, sw_ref, out_ref, gather_ref,
             send_sems, recv_sems):
        my = lax.axis_index("i")
        left = (my + N_DEV - 1) % N_DEV
        right = (my + 1) % N_DEV

        barrier_sem = pltpu.get_barrier_semaphore()
        for nbr in (left, right):
            pl.semaphore_signal(
                barrier_sem, inc=1,
                device_id=(nbr,), device_id_type=pl.DeviceIdType.MESH,
            )
        pl.semaphore_wait(barrier_sem, 2)

        scale = sx_ref[0] * sw_ref[0]

        def rows(origin):
            return pl.ds(origin * m_per, m_per)

        def block(origin):
            a = gather_ref[rows(origin), :]
            acc = lax.dot_general(
                a, w_ref[...],
                (((1,), (0,)), ((), ())),
                preferred_element_type=jnp.float32,
            )
            out_ref[rows(origin), :] = acc * scale

        gather_ref[rows(my), :] = x_ref[...]

        for h in range(N_DEV - 1):
            origin = (my - h) % N_DEV
            rdma = pltpu.make_async_remote_copy(
                src_ref=gather_ref.at[rows(origin), :],
                dst_ref=gather_ref.at[rows(origin), :],
                send_sem=send_sems.at[h],
                recv_sem=recv_sems.at[h],
                device_id=(right,),
                device_id_type=pl.DeviceIdType.MESH,
            )
            rdma.start()
            block(origin)
            rdma.wait()
        block((my + 1) % N_DEV)

    return pl.pallas_call(
        body,
        out_shape=jax.ShapeDtypeStruct((N_DEV * m_per, n_per), jnp.float32),
        in_specs=[
            pl.BlockSpec(memory_space=pltpu.VMEM),
            pl.BlockSpec(memory_space=pltpu.VMEM),
            pl.BlockSpec(memory_space=pltpu.SMEM),
            pl.BlockSpec(memory_space=pltpu.SMEM),
        ],
        out_specs=pl.BlockSpec(memory_space=pltpu.VMEM),
        scratch_shapes=[
            pltpu.VMEM((N_DEV * m_per, k), jnp.float8_e4m3fn),
            pltpu.SemaphoreType.DMA((N_DEV - 1,)),
            pltpu.SemaphoreType.DMA((N_DEV - 1,)),
        ],
        compiler_params=pltpu.CompilerParams(collective_id=0),
    )(x8, w8, sx, sw)


# baseline (device time: 217883 ns/iter reference)
import jax
import jax.numpy as jnp
from jax import lax
from jax.experimental import pallas as pl
from jax.experimental.pallas import tpu as pltpu

N_DEV = 4


def kernel(x, w_mat, scale_x, scale_w):
    m_per, k = x.shape
    _, n = w_mat.shape
    n_per = n // N_DEV

    me = lax.axis_index("i")
    w_blk = lax.dynamic_slice(w_mat, (0, me * n_per), (k, n_per))
    x8 = x.astype(jnp.float8_e4m3fn)
    w8 = w_blk.astype(jnp.float8_e4m3fn)
    sx = scale_x.astype(jnp.float32)
    sw = scale_w.astype(jnp.float32)

    def body(x_ref, w_ref, sx_ref, sw_ref, out_ref, gather_ref,
             send_sems, recv_sems):
        my = lax.axis_index("i")
        left = (my + N_DEV - 1) % N_DEV
        right = (my + 1) % N_DEV

        barrier_sem = pltpu.get_barrier_semaphore()
        for nbr in (left, right):
            pl.semaphore_signal(
                barrier_sem, inc=1,
                device_id=(nbr,), device_id_type=pl.DeviceIdType.MESH,
            )
        pl.semaphore_wait(barrier_sem, 2)

        scale = sx_ref[0] * sw_ref[0]

        def rows(origin):
            return pl.ds(origin * m_per, m_per)

        def block(origin):
            a = gather_ref[rows(origin), :]
            acc = lax.dot_general(
                a, w_ref[...],
                (((1,), (0,)), ((), ())),
                preferred_element_type=jnp.float32,
            )
            out_ref[rows(origin), :] = acc * scale

        gather_ref[rows(my), :] = x_ref[...]

        for h in range(N_DEV - 1):
            origin = (my - h) % N_DEV
            rdma = pltpu.make_async_remote_copy(
                src_ref=gather_ref.at[rows(origin), :],
                dst_ref=gather_ref.at[rows(origin), :],
                send_sem=send_sems.at[h],
                recv_sem=recv_sems.at[h],
                device_id=(right,),
                device_id_type=pl.DeviceIdType.MESH,
            )
            rdma.start()
            block(origin)
            rdma.wait()
        block((my + 1) % N_DEV)

    return pl.pallas_call(
        body,
        out_shape=jax.ShapeDtypeStruct((N_DEV * m_per, n_per), jnp.float32),
        in_specs=[
            pl.BlockSpec(memory_space=pltpu.VMEM),
            pl.BlockSpec(memory_space=pltpu.VMEM),
            pl.BlockSpec(memory_space=pltpu.SMEM),
            pl.BlockSpec(memory_space=pltpu.SMEM),
        ],
        out_specs=pl.BlockSpec(memory_space=pltpu.VMEM),
        scratch_shapes=[
            pltpu.VMEM((N_DEV * m_per, k), jnp.float8_e4m3fn),
            pltpu.SemaphoreType.DMA((N_DEV - 1,)),
            pltpu.SemaphoreType.DMA((N_DEV - 1,)),
        ],
        compiler_params=pltpu.CompilerParams(
            collective_id=0,
            vmem_limit_bytes=100 * 1024 * 1024,
        ),
    )(x8, w8, sx, sw)


# device time: 157893 ns/iter; 1.3799x vs baseline; 1.3799x over previous
import jax
import jax.numpy as jnp
from jax import lax
from jax.experimental import pallas as pl
from jax.experimental.pallas import tpu as pltpu

N_DEV = 4


def kernel(x, w_mat, scale_x, scale_w):
    m_per, k = x.shape
    _, n = w_mat.shape
    n_per = n // N_DEV
    half = m_per // 2

    me = lax.axis_index("i")
    w_blk = lax.dynamic_slice(w_mat, (0, me * n_per), (k, n_per))
    x8 = x.astype(jnp.float8_e4m3fn)
    w8 = w_blk.astype(jnp.float8_e4m3fn)
    sx = scale_x.astype(jnp.float32)
    sw = scale_w.astype(jnp.float32)

    def body(x_ref, w_ref, sx_ref, sw_ref, out_ref, gather_ref,
             send_sems, recv_sems):
        my = lax.axis_index("i")
        left = (my + N_DEV - 1) % N_DEV
        right = (my + 1) % N_DEV
        opp = (my + 2) % N_DEV

        barrier_sem = pltpu.get_barrier_semaphore()
        for nbr in (left, right):
            pl.semaphore_signal(
                barrier_sem, inc=1,
                device_id=(nbr,), device_id_type=pl.DeviceIdType.MESH,
            )
        pl.semaphore_wait(barrier_sem, 2)

        scale = sx_ref[0] * sw_ref[0]

        def rows(origin):
            return pl.ds(origin * m_per, m_per)

        def top(origin):
            return pl.ds(origin * m_per, half)

        def bot(origin):
            return pl.ds(origin * m_per + half, half)

        def block(origin):
            a = gather_ref[rows(origin), :]
            acc = lax.dot_general(
                a, w_ref[...],
                (((1,), (0,)), ((), ())),
                preferred_element_type=jnp.float32,
            )
            out_ref[rows(origin), :] = acc * scale

        def copy(src_sl, dst_sl, sem_i, target):
            return pltpu.make_async_remote_copy(
                src_ref=gather_ref.at[src_sl, :],
                dst_ref=gather_ref.at[dst_sl, :],
                send_sem=send_sems.at[sem_i],
                recv_sem=recv_sems.at[sem_i],
                device_id=(target,),
                device_id_type=pl.DeviceIdType.MESH,
            )

        gather_ref[rows(my), :] = x_ref[...]

        s_r0 = copy(rows(my), rows(my), 0, right)
        s_l0 = copy(rows(my), rows(my), 1, left)
        s_r0.start()
        s_l0.start()

        block(my)

        copy(rows(left), rows(left), 0, right).wait_recv()
        s_r1 = copy(top(left), top(left), 2, right)
        s_r1.start()
        block(left)

        copy(rows(right), rows(right), 1, left).wait_recv()
        s_l1 = copy(bot(right), bot(right), 3, left)
        s_l1.start()
        block(right)

        copy(top(opp), top(opp), 2, right).wait_recv()
        copy(bot(opp), bot(opp), 3, left).wait_recv()
        block(opp)

        s_r0.wait_send()
        s_l0.wait_send()
        s_r1.wait_send()
        s_l1.wait_send()

    return pl.pallas_call(
        body,
        out_shape=jax.ShapeDtypeStruct((N_DEV * m_per, n_per), jnp.float32),
        in_specs=[
            pl.BlockSpec(memory_space=pltpu.VMEM),
            pl.BlockSpec(memory_space=pltpu.VMEM),
            pl.BlockSpec(memory_space=pltpu.SMEM),
            pl.BlockSpec(memory_space=pltpu.SMEM),
        ],
        out_specs=pl.BlockSpec(memory_space=pltpu.VMEM),
        scratch_shapes=[
            pltpu.VMEM((N_DEV * m_per, k), jnp.float8_e4m3fn),
            pltpu.SemaphoreType.DMA((4,)),
            pltpu.SemaphoreType.DMA((4,)),
        ],
        compiler_params=pltpu.CompilerParams(
            collective_id=0,
            vmem_limit_bytes=100 * 1024 * 1024,
        ),
    )(x8, w8, sx, sw)


# device time: 108281 ns/iter; 2.0122x vs baseline; 1.4582x over previous
import jax
import jax.numpy as jnp
from jax import lax
from jax.experimental import pallas as pl
from jax.experimental.pallas import tpu as pltpu

N_DEV = 4
N_WCHUNK = 8


def kernel(x, w_mat, scale_x, scale_w):
    m_per, k = x.shape
    _, n = w_mat.shape
    n_per = n // N_DEV
    half = m_per // 2
    quart = m_per // 4
    wc = n_per // N_WCHUNK

    sx = scale_x.astype(jnp.float32)
    sw = scale_w.astype(jnp.float32)

    def body(x_ref, w_hbm, sx_ref, sw_ref, out_ref, gather_ref, w8_ref,
             wstage_ref, ostage_ref, wsems, osems, send_sems, recv_sems):
        my = lax.axis_index("i")
        left = (my + N_DEV - 1) % N_DEV
        right = (my + 1) % N_DEV
        opp = (my + 2) % N_DEV

        barrier_sem = pltpu.get_barrier_semaphore()
        for nbr in (left, right):
            pl.semaphore_signal(
                barrier_sem, inc=1,
                device_id=(nbr,), device_id_type=pl.DeviceIdType.MESH,
            )
        pl.semaphore_wait(barrier_sem, 2)

        scale = sx_ref[0] * sw_ref[0]

        def rows(origin, lo, nrows):
            return pl.ds(origin * m_per + lo, nrows)

        def copy(sl, sem_i, target):
            return pltpu.make_async_remote_copy(
                src_ref=gather_ref.at[sl, :],
                dst_ref=gather_ref.at[sl, :],
                send_sem=send_sems.at[sem_i],
                recv_sem=recv_sems.at[sem_i],
                device_id=(target,),
                device_id_type=pl.DeviceIdType.MESH,
            )

        gather_ref[rows(my, 0, half), :] = x_ref[0:half, :].astype(
            jnp.float8_e4m3fn)
        sends = [copy(rows(my, 0, half), 0, right),
                 copy(rows(my, 0, half), 2, left)]
        sends[0].start()
        sends[1].start()
        gather_ref[rows(my, half, half), :] = x_ref[half:m_per, :].astype(
            jnp.float8_e4m3fn)
        sends += [copy(rows(my, half, half), 1, right),
                  copy(rows(my, half, half), 3, left)]
        sends[2].start()
        sends[3].start()

        col0 = my * n_per

        def wdma(c, buf):
            return pltpu.make_async_copy(
                w_hbm.at[:, pl.ds(col0 + c * wc, wc)],
                wstage_ref.at[buf],
                wsems.at[buf],
            )

        wdma(0, 0).start()
        for c in range(N_WCHUNK):
            if c + 1 < N_WCHUNK:
                wdma(c + 1, (c + 1) % 2).start()
            wdma(c, c % 2).wait()
            w8_ref[:, pl.ds(c * wc, wc)] = wstage_ref[c % 2].astype(
                jnp.float8_e4m3fn)

        copy(rows(left, 0, half), 0, right).wait_recv()
        sends += [copy(rows(left, 0, quart), 4, right),
                  copy(rows(left, quart, quart), 5, right)]
        sends[4].start()
        sends[5].start()
        copy(rows(right, 0, half), 2, left).wait_recv()
        sends += [copy(rows(right, half, quart), 6, left),
                  copy(rows(right, half + quart, quart), 7, left)]
        sends[6].start()
        sends[7].start()

        pending = [None, None]

        def gemm_q(origin, q, slot):
            r = rows(origin, q * quart, quart)
            a = gather_ref[r, :]
            acc = lax.dot_general(
                a, w8_ref[...],
                (((1,), (0,)), ((), ())),
                preferred_element_type=jnp.float32,
            )
            if pending[slot] is not None:
                pending[slot].wait()
            ostage_ref[slot] = acc * scale
            cp = pltpu.make_async_copy(
                ostage_ref.at[slot], out_ref.at[r, :], osems.at[slot])
            cp.start()
            pending[slot] = cp

        order = [(my, 0), (my, 1), (my, 2), (my, 3)]
        slot = 0
        for origin, q in order:
            gemm_q(origin, q, slot)
            slot ^= 1

        def run(origin, qs):
            nonlocal slot
            for q in qs:
                gemm_q(origin, q, slot)
                slot ^= 1

        run(left, (0, 1))
        run(right, (0, 1))
        copy(rows(left, half, half), 1, right).wait_recv()
        run(left, (2, 3))
        copy(rows(right, half, half), 3, left).wait_recv()
        run(right, (2, 3))
        copy(rows(opp, 0, quart), 4, right).wait_recv()
        run(opp, (0,))
        copy(rows(opp, half, quart), 6, left).wait_recv()
        run(opp, (2,))
        copy(rows(opp, quart, quart), 5, right).wait_recv()
        run(opp, (1,))
        copy(rows(opp, half + quart, quart), 7, left).wait_recv()
        run(opp, (3,))

        for cp in pending:
            if cp is not None:
                cp.wait()
        for s in sends:
            s.wait_send()

    return pl.pallas_call(
        body,
        out_shape=jax.ShapeDtypeStruct((N_DEV * m_per, n_per), jnp.float32),
        in_specs=[
            pl.BlockSpec(memory_space=pltpu.VMEM),
            pl.BlockSpec(memory_space=pltpu.MemorySpace.HBM),
            pl.BlockSpec(memory_space=pltpu.SMEM),
            pl.BlockSpec(memory_space=pltpu.SMEM),
        ],
        out_specs=pl.BlockSpec(memory_space=pltpu.MemorySpace.HBM),
        scratch_shapes=[
            pltpu.VMEM((N_DEV * m_per, k), jnp.float8_e4m3fn),
            pltpu.VMEM((k, n_per), jnp.float8_e4m3fn),
            pltpu.VMEM((2, k, n_per // N_WCHUNK), jnp.float32),
            pltpu.VMEM((2, m_per // 4, n_per), jnp.float32),
            pltpu.SemaphoreType.DMA((2,)),
            pltpu.SemaphoreType.DMA((2,)),
            pltpu.SemaphoreType.DMA((8,)),
            pltpu.SemaphoreType.DMA((8,)),
        ],
        compiler_params=pltpu.CompilerParams(
            collective_id=0,
            vmem_limit_bytes=60 * 1024 * 1024,
        ),
    )(x, w_mat, sx, sw)


# device time: 98265 ns/iter; 2.2173x vs baseline; 1.1019x over previous
import jax
import jax.numpy as jnp
from jax import lax
from jax.experimental import pallas as pl
from jax.experimental.pallas import tpu as pltpu

N_DEV = 4
N_WCHUNK = 8


def kernel(x, w_mat, scale_x, scale_w):
    m_per, k = x.shape
    _, n = w_mat.shape
    n_per = n // N_DEV
    half = m_per // 2
    quart = m_per // 4
    wc = n_per // N_WCHUNK

    sx = scale_x.astype(jnp.float32)
    sw = scale_w.astype(jnp.float32)

    def body(x_ref, w_hbm, sx_ref, sw_ref, out_ref, gather_ref, w8_ref,
             wstage_ref, ostage_ref, wsems, osems, send_sems, recv_sems):
        my = lax.axis_index("i")
        left = (my + N_DEV - 1) % N_DEV
        right = (my + 1) % N_DEV
        opp = (my + 2) % N_DEV

        barrier_sem = pltpu.get_barrier_semaphore()
        for nbr in (left, right):
            pl.semaphore_signal(
                barrier_sem, inc=1,
                device_id=(nbr,), device_id_type=pl.DeviceIdType.MESH,
            )
        pl.semaphore_wait(barrier_sem, 2)

        scale = sx_ref[0] * sw_ref[0]

        def rows(origin, lo, nrows):
            return pl.ds(origin * m_per + lo, nrows)

        def copy(sl, sem_i, target):
            return pltpu.make_async_remote_copy(
                src_ref=gather_ref.at[sl, :],
                dst_ref=gather_ref.at[sl, :],
                send_sem=send_sems.at[sem_i],
                recv_sem=recv_sems.at[sem_i],
                device_id=(target,),
                device_id_type=pl.DeviceIdType.MESH,
            )

        gather_ref[rows(my, 0, half), :] = x_ref[0:half, :].astype(
            jnp.float8_e4m3fn)
        sends = [copy(rows(my, 0, half), 0, right),
                 copy(rows(my, 0, half), 2, left)]
        sends[0].start()
        sends[1].start()
        gather_ref[rows(my, half, half), :] = x_ref[half:m_per, :].astype(
            jnp.float8_e4m3fn)
        sends += [copy(rows(my, half, half), 1, right),
                  copy(rows(my, half, half), 3, left)]
        sends[2].start()
        sends[3].start()

        col0 = my * n_per

        def wdma(c, buf):
            return pltpu.make_async_copy(
                w_hbm.at[:, pl.ds(col0 + c * wc, wc)],
                wstage_ref.at[buf],
                wsems.at[buf],
            )

        wdma(0, 0).start()
        for c in range(N_WCHUNK):
            if c + 1 < N_WCHUNK:
                wdma(c + 1, (c + 1) % 2).start()
            wdma(c, c % 2).wait()
            w8_ref[:, pl.ds(c * wc, wc)] = wstage_ref[c % 2].astype(
                jnp.float8_e4m3fn)

        p1 = 3 * m_per // 8
        p2 = half - p1
        copy(rows(left, 0, half), 0, right).wait_recv()
        sends += [copy(rows(left, 0, p1), 4, right),
                  copy(rows(left, p1, p2), 5, right)]
        sends[4].start()
        sends[5].start()
        copy(rows(right, 0, half), 2, left).wait_recv()
        sends += [copy(rows(right, half, p1), 6, left),
                  copy(rows(right, half + p1, p2), 7, left)]
        sends[6].start()
        sends[7].start()

        pending = [None, None]
        slot = 0

        def gemm_rows(origin, lo, nrows):
            nonlocal slot
            r = rows(origin, lo, nrows)
            a = gather_ref[r, :]
            acc = lax.dot_general(
                a, w8_ref[...],
                (((1,), (0,)), ((), ())),
                preferred_element_type=jnp.float32,
            )
            if pending[slot] is not None:
                pending[slot].wait()
            ostage_ref[slot, 0:nrows, :] = (acc * scale).astype(jnp.bfloat16)
            cp = pltpu.make_async_copy(
                ostage_ref.at[slot, pl.ds(0, nrows), :], out_ref.at[r, :],
                osems.at[slot])
            cp.start()
            pending[slot] = cp
            slot ^= 1

        for q in range(4):
            gemm_rows(my, q * quart, quart)
        for q in range(2):
            gemm_rows(left, q * quart, quart)
        for q in range(2):
            gemm_rows(right, q * quart, quart)
        copy(rows(left, half, half), 1, right).wait_recv()
        for q in range(2, 4):
            gemm_rows(left, q * quart, quart)
        copy(rows(right, half, half), 3, left).wait_recv()
        for q in range(2, 4):
            gemm_rows(right, q * quart, quart)
        copy(rows(opp, 0, p1), 4, right).wait_recv()
        gemm_rows(opp, 0, p1)
        copy(rows(opp, half, p1), 6, left).wait_recv()
        gemm_rows(opp, half, p1)
        copy(rows(opp, p1, p2), 5, right).wait_recv()
        gemm_rows(opp, p1, p2)
        copy(rows(opp, half + p1, p2), 7, left).wait_recv()
        gemm_rows(opp, half + p1, p2)

        for cp in pending:
            if cp is not None:
                cp.wait()
        for s in sends:
            s.wait_send()

    return pl.pallas_call(
        body,
        out_shape=jax.ShapeDtypeStruct((N_DEV * m_per, n_per), jnp.bfloat16),
        in_specs=[
            pl.BlockSpec(memory_space=pltpu.VMEM),
            pl.BlockSpec(memory_space=pltpu.MemorySpace.HBM),
            pl.BlockSpec(memory_space=pltpu.SMEM),
            pl.BlockSpec(memory_space=pltpu.SMEM),
        ],
        out_specs=pl.BlockSpec(memory_space=pltpu.MemorySpace.HBM),
        scratch_shapes=[
            pltpu.VMEM((N_DEV * m_per, k), jnp.float8_e4m3fn),
            pltpu.VMEM((k, n_per), jnp.float8_e4m3fn),
            pltpu.VMEM((2, k, n_per // N_WCHUNK), jnp.float32),
            pltpu.VMEM((2, 3 * m_per // 8, n_per), jnp.bfloat16),
            pltpu.SemaphoreType.DMA((2,)),
            pltpu.SemaphoreType.DMA((2,)),
            pltpu.SemaphoreType.DMA((8,)),
            pltpu.SemaphoreType.DMA((8,)),
        ],
        compiler_params=pltpu.CompilerParams(
            collective_id=0,
            vmem_limit_bytes=60 * 1024 * 1024,
        ),
    )(x, w_mat, sx, sw)


# device time: 95083 ns/iter; 2.2915x vs baseline; 1.0335x over previous
import jax
import jax.numpy as jnp
from jax import lax
from jax.experimental import pallas as pl
from jax.experimental.pallas import tpu as pltpu

N_DEV = 4
N_WCHUNK = 8


def kernel(x, w_mat, scale_x, scale_w):
    m_per, k = x.shape
    _, n = w_mat.shape
    n_per = n // N_DEV
    half = m_per // 2
    quart = m_per // 4
    wc = n_per // N_WCHUNK

    sx = scale_x.astype(jnp.float32)
    sw = scale_w.astype(jnp.float32)

    def body(x_hbm, w_hbm, sx_ref, sw_ref, out_ref, gather_ref, w8_ref,
             wstage_ref, ostage_ref, xstage_ref, wsems, osems, xsems,
             send_sems, recv_sems):
        my = lax.axis_index("i")
        left = (my + N_DEV - 1) % N_DEV
        right = (my + 1) % N_DEV
        opp = (my + 2) % N_DEV

        xdmas = [
            pltpu.make_async_copy(
                x_hbm.at[pl.ds(h * half, half), :], xstage_ref.at[h],
                xsems.at[h])
            for h in range(2)
        ]
        xdmas[0].start()
        xdmas[1].start()

        barrier_sem = pltpu.get_barrier_semaphore()
        for nbr in (left, right):
            pl.semaphore_signal(
                barrier_sem, inc=1,
                device_id=(nbr,), device_id_type=pl.DeviceIdType.MESH,
            )
        pl.semaphore_wait(barrier_sem, 2)

        scale = sx_ref[0] * sw_ref[0]

        def rows(origin, lo, nrows):
            return pl.ds(origin * m_per + lo, nrows)

        def copy(sl, sem_i, target):
            return pltpu.make_async_remote_copy(
                src_ref=gather_ref.at[sl, :],
                dst_ref=gather_ref.at[sl, :],
                send_sem=send_sems.at[sem_i],
                recv_sem=recv_sems.at[sem_i],
                device_id=(target,),
                device_id_type=pl.DeviceIdType.MESH,
            )

        xdmas[0].wait()
        gather_ref[rows(my, 0, half), :] = xstage_ref[0].astype(
            jnp.float8_e4m3fn)
        sends = [copy(rows(my, 0, half), 0, right),
                 copy(rows(my, 0, half), 2, left)]
        sends[0].start()
        sends[1].start()
        xdmas[1].wait()
        gather_ref[rows(my, half, half), :] = xstage_ref[1].astype(
            jnp.float8_e4m3fn)
        sends += [copy(rows(my, half, half), 1, right),
                  copy(rows(my, half, half), 3, left)]
        sends[2].start()
        sends[3].start()

        col0 = my * n_per

        def wdma(c, buf):
            return pltpu.make_async_copy(
                w_hbm.at[:, pl.ds(col0 + c * wc, wc)],
                wstage_ref.at[buf],
                wsems.at[buf],
            )

        wdma(0, 0).start()
        for c in range(N_WCHUNK):
            if c + 1 < N_WCHUNK:
                wdma(c + 1, (c + 1) % 2).start()
            wdma(c, c % 2).wait()
            w8_ref[:, pl.ds(c * wc, wc)] = wstage_ref[c % 2].astype(
                jnp.float8_e4m3fn)

        p1 = 3 * m_per // 8
        p2 = half - p1
        copy(rows(left, 0, half), 0, right).wait_recv()
        sends += [copy(rows(left, 0, p1), 4, right),
                  copy(rows(left, p1, p2), 5, right)]
        sends[4].start()
        sends[5].start()
        copy(rows(right, 0, half), 2, left).wait_recv()
        sends += [copy(rows(right, half, p1), 6, left),
                  copy(rows(right, half + p1, p2), 7, left)]
        sends[6].start()
        sends[7].start()

        pending = [None, None]
        slot = 0

        def gemm_rows(origin, lo, nrows):
            nonlocal slot
            r = rows(origin, lo, nrows)
            a = gather_ref[r, :]
            acc = lax.dot_general(
                a, w8_ref[...],
                (((1,), (0,)), ((), ())),
                preferred_element_type=jnp.float32,
            )
            if pending[slot] is not None:
                pending[slot].wait()
            ostage_ref[slot, 0:nrows, :] = (acc * scale).astype(jnp.bfloat16)
            cp = pltpu.make_async_copy(
                ostage_ref.at[slot, pl.ds(0, nrows), :], out_ref.at[r, :],
                osems.at[slot])
            cp.start()
            pending[slot] = cp
            slot ^= 1

        for q in range(4):
            gemm_rows(my, q * quart, quart)
        for q in range(2):
            gemm_rows(left, q * quart, quart)
        for q in range(2):
            gemm_rows(right, q * quart, quart)
        copy(rows(left, half, half), 1, right).wait_recv()
        for q in range(2, 4):
            gemm_rows(left, q * quart, quart)
        copy(rows(right, half, half), 3, left).wait_recv()
        for q in range(2, 4):
            gemm_rows(right, q * quart, quart)
        copy(rows(opp, 0, p1), 4, right).wait_recv()
        gemm_rows(opp, 0, p1)
        copy(rows(opp, half, p1), 6, left).wait_recv()
        gemm_rows(opp, half, p1)
        copy(rows(opp, p1, p2), 5, right).wait_recv()
        gemm_rows(opp, p1, p2)
        copy(rows(opp, half + p1, p2), 7, left).wait_recv()
        gemm_rows(opp, half + p1, p2)

        for cp in pending:
            if cp is not None:
                cp.wait()
        for s in sends:
            s.wait_send()

    return pl.pallas_call(
        body,
        out_shape=jax.ShapeDtypeStruct((N_DEV * m_per, n_per), jnp.bfloat16),
        in_specs=[
            pl.BlockSpec(memory_space=pltpu.MemorySpace.HBM),
            pl.BlockSpec(memory_space=pltpu.MemorySpace.HBM),
            pl.BlockSpec(memory_space=pltpu.SMEM),
            pl.BlockSpec(memory_space=pltpu.SMEM),
        ],
        out_specs=pl.BlockSpec(memory_space=pltpu.MemorySpace.HBM),
        scratch_shapes=[
            pltpu.VMEM((N_DEV * m_per, k), jnp.float8_e4m3fn),
            pltpu.VMEM((k, n_per), jnp.float8_e4m3fn),
            pltpu.VMEM((2, k, n_per // N_WCHUNK), jnp.float32),
            pltpu.VMEM((2, 3 * m_per // 8, n_per), jnp.bfloat16),
            pltpu.VMEM((2, m_per // 2, k), jnp.float32),
            pltpu.SemaphoreType.DMA((2,)),
            pltpu.SemaphoreType.DMA((2,)),
            pltpu.SemaphoreType.DMA((2,)),
            pltpu.SemaphoreType.DMA((8,)),
            pltpu.SemaphoreType.DMA((8,)),
        ],
        compiler_params=pltpu.CompilerParams(
            collective_id=0,
            vmem_limit_bytes=60 * 1024 * 1024,
        ),
    )(x, w_mat, sx, sw)
